# baseline (speedup 1.0000x reference)
import functools

import jax
import jax.numpy as jnp
from jax import lax
from jax.experimental import pallas as pl
from jax.experimental.pallas import tpu as pltpu
from jax.experimental.pallas import tpu_sc as plsc

N = 10000
E = 320000
D = 128
DW = D // 2
EPS = 1e-5

NC = 2
NS = 16
NW = NC * NS
EPW = E // NW
KG = 80
NCG = EPW // KG
KS = 80
NCS = EPW // KS
NPT = 624
NTAIL = N - NS * NPT

_HIMASK = -65536


def _pack_rows(y):
    u = jax.lax.bitcast_convert_type(y, jnp.int32)
    r = u + 0x7FFF + (jax.lax.shift_right_logical(u, 16) & 1)
    top = jax.lax.shift_right_logical(r, 16)
    return top[:, :DW] | jax.lax.shift_left(top[:, DW:], 16)


def _unpack_rows(hw):
    lo = jax.lax.bitcast_convert_type(jax.lax.shift_left(hw, 16), jnp.float32)
    hi = jax.lax.bitcast_convert_type(hw & _HIMASK, jnp.float32)
    return jnp.concatenate([lo, hi], axis=1)


def _y_body(x_ref, w1_ref, b1_ref, y1_ref, y2_ref):
    x = x_ref[...]
    wb = w1_ref[128:256, :]
    wd = w1_ref[0:128, :] - wb
    y1_ref[...] = jnp.dot(x, wd, preferred_element_type=jnp.float32) + b1_ref[...]
    y2_ref[...] = jnp.dot(x, wb, preferred_element_type=jnp.float32)


def _mlp1_body(h_ref, s_ref, t_ref, w_ref, b_ref, o_ref, st_ref):
    h = _unpack_rows(h_ref[...])
    a = jnp.maximum(h * s_ref[...] + t_ref[...], 0.0)
    hn = jnp.dot(a.astype(jnp.bfloat16), w_ref[...],
                 preferred_element_type=jnp.float32) + b_ref[...]
    o_ref[...] = hn.astype(jnp.bfloat16)

    @pl.when(pl.program_id(0) == 0)
    def _init():
        st_ref[...] = jnp.zeros_like(st_ref)

    su = jnp.sum(hn, axis=0, keepdims=True)
    sq = jnp.sum(hn * hn, axis=0, keepdims=True)
    pad = jnp.zeros((6, D), jnp.float32)
    st_ref[...] = st_ref[...] + jnp.concatenate([su, sq, pad], axis=0)


def _mlp2_body(h_ref, s_ref, t_ref, w_ref, b_ref, o_ref, st_ref):
    h = h_ref[...].astype(jnp.float32)
    a = jnp.maximum(h * s_ref[...] + t_ref[...], 0.0)
    hn = jnp.dot(a.astype(jnp.bfloat16), w_ref[...],
                 preferred_element_type=jnp.float32) + b_ref[...]
    o_ref[...] = _pack_rows(hn)

    @pl.when(pl.program_id(0) == 0)
    def _init():
        st_ref[...] = jnp.zeros_like(st_ref)

    su = jnp.sum(hn, axis=0, keepdims=True)
    sq = jnp.sum(hn * hn, axis=0, keepdims=True)
    pad = jnp.zeros((6, D), jnp.float32)
    st_ref[...] = st_ref[...] + jnp.concatenate([su, sq, pad], axis=0)


def _add_body(p_ref, o_ref):
    o_ref[...] = p_ref[0] + p_ref[1]


_BE = 4000
_BN0 = 2000
_BA = 2000


def _y_call(x, w1, b1r):
    return pl.pallas_call(
        _y_body,
        grid=(N // _BN0,),
        in_specs=[
            pl.BlockSpec((_BN0, D), lambda i: (i, 0)),
            pl.BlockSpec((2 * D, D), lambda i: (0, 0)),
            pl.BlockSpec((1, D), lambda i: (0, 0)),
        ],
        out_specs=[
            pl.BlockSpec((_BN0, D), lambda i: (i, 0)),
            pl.BlockSpec((_BN0, D), lambda i: (i, 0)),
        ],
        out_shape=[
            jax.ShapeDtypeStruct((N, D), jnp.float32),
            jax.ShapeDtypeStruct((N, D), jnp.float32),
        ],
    )(x, w1, b1r)


def _mlp1_call(h, s, t, w, b):
    return pl.pallas_call(
        _mlp1_body,
        grid=(E // _BE,),
        in_specs=[
            pl.BlockSpec((_BE, DW), lambda i: (i, 0)),
            pl.BlockSpec((1, D), lambda i: (0, 0)),
            pl.BlockSpec((1, D), lambda i: (0, 0)),
            pl.BlockSpec((D, D), lambda i: (0, 0)),
            pl.BlockSpec((1, D), lambda i: (0, 0)),
        ],
        out_specs=[
            pl.BlockSpec((_BE, D), lambda i: (i, 0)),
            pl.BlockSpec((8, D), lambda i: (0, 0)),
        ],
        out_shape=[
            jax.ShapeDtypeStruct((E, D), jnp.bfloat16),
            jax.ShapeDtypeStruct((8, D), jnp.float32),
        ],
    )(h, s, t, w, b)


def _mlp2_call(h, s, t, w, b):
    return pl.pallas_call(
        _mlp2_body,
        grid=(E // _BE,),
        in_specs=[
            pl.BlockSpec((_BE, D), lambda i: (i, 0)),
            pl.BlockSpec((1, D), lambda i: (0, 0)),
            pl.BlockSpec((1, D), lambda i: (0, 0)),
            pl.BlockSpec((D, D), lambda i: (0, 0)),
            pl.BlockSpec((1, D), lambda i: (0, 0)),
        ],
        out_specs=[
            pl.BlockSpec((_BE, DW), lambda i: (i, 0)),
            pl.BlockSpec((8, D), lambda i: (0, 0)),
        ],
        out_shape=[
            jax.ShapeDtypeStruct((E, DW), jnp.int32),
            jax.ShapeDtypeStruct((8, D), jnp.float32),
        ],
    )(h, s, t, w, b)


def _add_call(parts):
    return pl.pallas_call(
        _add_body,
        grid=(N // _BA,),
        in_specs=[pl.BlockSpec((2, _BA, D), lambda i: (0, i, 0))],
        out_specs=pl.BlockSpec((_BA, D), lambda i: (i, 0)),
        out_shape=jax.ShapeDtypeStruct((N, D), jnp.float32),
    )(parts)


def _gather_body(y1_hbm, y2_hbm, dst_hbm, src_hbm, h1_hbm, st_hbm,
                 idx_d, idx_s, rows_d0, rows_d1, rows_s0, rows_s1,
                 hbuf0, hbuf1, statbuf,
                 semd0, semd1, sems0, sems1, semw0, semw1):
    cid = lax.axis_index("c")
    sid = lax.axis_index("s")
    wid = sid * NC + cid
    base = wid * EPW
    rows_d = (rows_d0, rows_d1)
    rows_s = (rows_s0, rows_s1)
    hbuf = (hbuf0, hbuf1)
    semd = (semd0, semd1)
    sems = (sems0, sems1)
    semw = (semw0, semw1)

    pltpu.sync_copy(dst_hbm.at[pl.ds(base, EPW)], idx_d)
    pltpu.sync_copy(src_hbm.at[pl.ds(base, EPW)], idx_s)
    for j in range(16):
        statbuf[pl.ds(j * 16, 16)] = jnp.zeros((16,), jnp.float32)

    def start_gather(b, c):
        off = c * KG
        pltpu.async_copy(y1_hbm.at[idx_d.at[pl.ds(off, KG)]], rows_d[b], semd[b])
        pltpu.async_copy(y2_hbm.at[idx_s.at[pl.ds(off, KG)]], rows_s[b], sems[b])

    start_gather(0, 0)
    start_gather(1, 1)

    def chunk_body(g, _):
        for b in range(2):
            c = 2 * g + b

            @pl.when(c < NCG)
            def _do():
                off = c * KG
                pltpu.make_async_copy(y1_hbm.at[idx_d.at[pl.ds(off, KG)]],
                                      rows_d[b], semd[b]).wait()
                pltpu.make_async_copy(y2_hbm.at[idx_s.at[pl.ds(off, KG)]],
                                      rows_s[b], sems[b]).wait()

                @pl.when(c >= 2)
                def _drain_write():
                    pltpu.make_async_copy(
                        hbuf[b], h1_hbm.at[pl.ds(base + (c - 2) * KG, KG)],
                        semw[b]).wait()

                def row_body(r, a):
                    bc = jax.lax.bitcast_convert_type
                    hs = []
                    sums = []
                    sqs = []
                    for j in range(8):
                        dv = rows_d[b][r, pl.ds(j * 16, 16)]
                        sv = rows_s[b][r, pl.ds(j * 16, 16)]
                        h = dv + sv
                        hs.append(h)
                        sums.append(a[j] + h)
                        sqs.append(a[8 + j] + h * h)
                    for j in range(4):
                        ulo = bc(hs[j], jnp.int32)
                        uhi = bc(hs[4 + j], jnp.int32)
                        hbuf[b][r, pl.ds(j * 16, 16)] = (
                            jax.lax.shift_right_logical(ulo, 16)
                            | (uhi & _HIMASK))
                    return tuple(sums + sqs)

                acc0 = tuple(jnp.zeros((16,), jnp.float32) for _ in range(16))
                acc = lax.fori_loop(0, KG, row_body, acc0)
                for j in range(16):
                    statbuf[pl.ds(j * 16, 16)] = (
                        statbuf[pl.ds(j * 16, 16)] + acc[j])

                pltpu.async_copy(hbuf[b], h1_hbm.at[pl.ds(base + c * KG, KG)],
                                 semw[b])

                @pl.when(c + 2 < NCG)
                def _next():
                    start_gather(b, c + 2)

        return 0

    lax.fori_loop(0, (NCG + 1) // 2, chunk_body, 0)
    pltpu.make_async_copy(hbuf[0], h1_hbm.at[pl.ds(base + (NCG - 1) * KG, KG)],
                          semw[0]).wait()
    pltpu.make_async_copy(hbuf[1], h1_hbm.at[pl.ds(base + (NCG - 2) * KG, KG)],
                          semw[1]).wait()
    pltpu.sync_copy(statbuf, st_hbm.at[wid])


def _gather_call(y1, y2, dst, src):
    mesh = plsc.VectorSubcoreMesh(core_axis_name="c", subcore_axis_name="s")
    f = functools.partial(
        pl.kernel,
        mesh=mesh,
        out_type=[
            jax.ShapeDtypeStruct((E, DW), jnp.int32),
            jax.ShapeDtypeStruct((NW, 2 * D), jnp.float32),
        ],
        scratch_types=[
            pltpu.VMEM((EPW,), jnp.int32),
            pltpu.VMEM((EPW,), jnp.int32),
            pltpu.VMEM((KG, D), jnp.float32),
            pltpu.VMEM((KG, D), jnp.float32),
            pltpu.VMEM((KG, D), jnp.float32),
            pltpu.VMEM((KG, D), jnp.float32),
            pltpu.VMEM((KG, DW), jnp.int32),
            pltpu.VMEM((KG, DW), jnp.int32),
            pltpu.VMEM((2 * D,), jnp.float32),
            pltpu.SemaphoreType.DMA,
            pltpu.SemaphoreType.DMA,
            pltpu.SemaphoreType.DMA,
            pltpu.SemaphoreType.DMA,
            pltpu.SemaphoreType.DMA,
            pltpu.SemaphoreType.DMA,
        ],
    )(_gather_body)
    return f(y1, y2, dst, src)


def _scatter_body(h3_hbm, d1_hbm, s_hbm, t_hbm, z_hbm, out_hbm,
                  rbuf0, rbuf1, wbuf0, wbuf1,
                  ib00, ib01, ib10, ib11, sbuf, tbuf, acc_shared,
                  semr0, semr1, semw0, semw1,
                  semi00, semi01, semi10, semi11):
    cid = lax.axis_index("c")
    sid = lax.axis_index("s")
    wid = sid * NC + cid
    ebase = wid * EPW
    rbuf = (rbuf0, rbuf1)
    wbuf = (wbuf0, wbuf1)
    ibuf = ((ib00, ib01), (ib10, ib11))
    semr = (semr0, semr1)
    semw = (semw0, semw1)
    semi = ((semi00, semi01), (semi10, semi11))

    pltpu.sync_copy(s_hbm, sbuf)
    pltpu.sync_copy(t_hbm, tbuf)
    pltpu.sync_copy(z_hbm.at[pl.ds(sid * NPT, NPT)],
                    acc_shared.at[pl.ds(sid * NPT, NPT)])

    @pl.when(sid == NS - 1)
    def _zero_tail():
        pltpu.sync_copy(z_hbm.at[pl.ds(NS * NPT, NTAIL)],
                        acc_shared.at[pl.ds(NS * NPT, NTAIL)])

    svs = [sbuf[pl.ds(j * 16, 16)] for j in range(8)]
    tvs = [tbuf[pl.ds(j * 16, 16)] for j in range(8)]

    def start_read(b, il, c):
        off = ebase + c * KS
        pltpu.async_copy(h3_hbm.at[pl.ds(off, KS)], rbuf[b], semr[b])
        pltpu.async_copy(d1_hbm.at[pl.ds(off, KS)], ibuf[b][il], semi[b][il])

    start_read(0, 0, 0)
    start_read(1, 0, 1)
    plsc.subcore_barrier()

    def chunk_body(g, _):
        for q in range(4):
            c = 4 * g + q
            b = q % 2
            il = q // 2

            @pl.when(c < NCS)
            def _do():
                pltpu.make_async_copy(h3_hbm.at[pl.ds(0, KS)], rbuf[b],
                                      semr[b]).wait()
                pltpu.make_async_copy(d1_hbm.at[pl.ds(0, KS)], ibuf[b][il],
                                      semi[b][il]).wait()

                @pl.when(c >= 2)
                def _drain_add():
                    pltpu.make_async_copy(wbuf[b],
                                          acc_shared.at[ibuf[b][1 - il]],
                                          semw[b]).wait()

                def row_body(r, rr):
                    bc = jax.lax.bitcast_convert_type
                    for j in range(4):
                        v = rbuf[b][r, pl.ds(j * 16, 16)]
                        lo = bc(jax.lax.shift_left(v, 16), jnp.float32)
                        hi = bc(v & _HIMASK, jnp.float32)
                        wbuf[b][r, pl.ds(j * 16, 16)] = jnp.maximum(
                            lo * svs[j] + tvs[j], 0.0)
                        wbuf[b][r, pl.ds((4 + j) * 16, 16)] = jnp.maximum(
                            hi * svs[4 + j] + tvs[4 + j], 0.0)
                    return rr

                lax.fori_loop(0, KS, row_body, 0)
                pltpu.async_copy(wbuf[b], acc_shared.at[ibuf[b][il]],
                                 semw[b], add=True)

                @pl.when(c + 2 < NCS)
                def _next():
                    start_read(b, 1 - il, c + 2)

        return 0

    lax.fori_loop(0, (NCS + 3) // 4, chunk_body, 0)
    pltpu.make_async_copy(wbuf[0], acc_shared.at[ibuf[0][0]], semw[0]).wait()
    pltpu.make_async_copy(wbuf[1], acc_shared.at[ibuf[1][1]], semw[1]).wait()
    plsc.subcore_barrier()
    pltpu.sync_copy(acc_shared.at[pl.ds(sid * NPT, NPT)],
                    out_hbm.at[cid, pl.ds(sid * NPT, NPT)])

    @pl.when(sid == NS - 1)
    def _dump_tail():
        pltpu.sync_copy(acc_shared.at[pl.ds(NS * NPT, NTAIL)],
                        out_hbm.at[cid, pl.ds(NS * NPT, NTAIL)])


def _scatter_call(h3, dst, s3, t3, zeros_nd):
    mesh = plsc.VectorSubcoreMesh(core_axis_name="c", subcore_axis_name="s")
    f = functools.partial(
        pl.kernel,
        mesh=mesh,
        out_type=jax.ShapeDtypeStruct((NC, N, D), jnp.float32),
        scratch_types=[
            pltpu.VMEM((KS, DW), jnp.int32),
            pltpu.VMEM((KS, DW), jnp.int32),
            pltpu.VMEM((KS, D), jnp.float32),
            pltpu.VMEM((KS, D), jnp.float32),
            pltpu.VMEM((KS,), jnp.int32),
            pltpu.VMEM((KS,), jnp.int32),
            pltpu.VMEM((KS,), jnp.int32),
            pltpu.VMEM((KS,), jnp.int32),
            pltpu.VMEM((D,), jnp.float32),
            pltpu.VMEM((D,), jnp.float32),
            pltpu.VMEM_SHARED((N, D), jnp.float32),
            pltpu.SemaphoreType.DMA,
            pltpu.SemaphoreType.DMA,
            pltpu.SemaphoreType.DMA,
            pltpu.SemaphoreType.DMA,
            pltpu.SemaphoreType.DMA,
            pltpu.SemaphoreType.DMA,
            pltpu.SemaphoreType.DMA,
            pltpu.SemaphoreType.DMA,
        ],
    )(_scatter_body)
    return f(h3, dst, s3, t3, zeros_nd)


def _affine(su, sq, g, be):
    m = su / E
    v = sq / E - m * m
    s = g * lax.rsqrt(v + EPS)
    t = be - m * s
    return s, t


def kernel(X, edge_index, W1, b1, g1, be1, W2, b2, g2, be2, W3, b3, g3, be3):
    ei = edge_index.astype(jnp.int32)
    src = ei[0]
    dst = ei[1]

    y1, y2 = _y_call(X, W1, b1.reshape(1, D))

    h1, st1p = _gather_call(y1, y2, dst, src)
    p = st1p.reshape(NW, 2, D)
    s1, t1 = _affine(jnp.sum(p[:, 0, :], axis=0),
                     jnp.sum(p[:, 1, :], axis=0), g1, be1)

    h2, st2 = _mlp1_call(h1, s1.reshape(1, D), t1.reshape(1, D),
                         W2.astype(jnp.bfloat16), b2.reshape(1, D))
    s2, t2 = _affine(st2[0], st2[1], g2, be2)

    h3, st3 = _mlp2_call(h2, s2.reshape(1, D), t2.reshape(1, D),
                         W3.astype(jnp.bfloat16), b3.reshape(1, D))
    s3, t3 = _affine(st3[0], st3[1], g3, be3)

    parts = _scatter_call(h3, dst, s3, t3, jnp.zeros((N, D), jnp.float32))
    return _add_call(parts)


# speedup vs baseline: 1.0052x; 1.0052x over previous
import functools

import jax
import jax.numpy as jnp
from jax import lax
from jax.experimental import pallas as pl
from jax.experimental.pallas import tpu as pltpu
from jax.experimental.pallas import tpu_sc as plsc

N = 10000
E = 320000
D = 128
DW = D // 2
EPS = 1e-5

NC = 2
NS = 16
NW = NC * NS
EPW = E // NW
KG = 80
NCG = EPW // KG
KS = 80
NCS = EPW // KS
NPT = 624
NTAIL = N - NS * NPT

_HIMASK = -65536


def _pack_rows(y):
    u = jax.lax.bitcast_convert_type(y, jnp.int32)
    r = u + 0x7FFF + (jax.lax.shift_right_logical(u, 16) & 1)
    top = jax.lax.shift_right_logical(r, 16)
    return top[:, :DW] | jax.lax.shift_left(top[:, DW:], 16)


def _unpack_rows(hw):
    lo = jax.lax.bitcast_convert_type(jax.lax.shift_left(hw, 16), jnp.float32)
    hi = jax.lax.bitcast_convert_type(hw & _HIMASK, jnp.float32)
    return jnp.concatenate([lo, hi], axis=1)


def _y_body(x_ref, w1_ref, b1_ref, y1_ref, y2_ref):
    x = x_ref[...]
    wb = w1_ref[128:256, :]
    wd = w1_ref[0:128, :] - wb
    y1_ref[...] = jnp.dot(x, wd, preferred_element_type=jnp.float32) + b1_ref[...]
    y2_ref[...] = jnp.dot(x, wb, preferred_element_type=jnp.float32)


def _affine_rows(su, sq, g, be):
    m = su * (1.0 / E)
    v = sq * (1.0 / E) - m * m
    s = g * lax.rsqrt(v + EPS)
    t = be - m * s
    return s, t


def _mlp1_body(h_ref, stp_ref, g_ref, be_ref, w_ref, b_ref, o_ref, st_ref):
    stp = stp_ref[...]
    su = jnp.sum(stp[:, :D], axis=0, keepdims=True)
    sq = jnp.sum(stp[:, D:], axis=0, keepdims=True)
    s, t = _affine_rows(su, sq, g_ref[...], be_ref[...])
    h = _unpack_rows(h_ref[...])
    a = jnp.maximum(h * s + t, 0.0)
    hn = jnp.dot(a.astype(jnp.bfloat16), w_ref[...].astype(jnp.bfloat16),
                 preferred_element_type=jnp.float32) + b_ref[...]
    o_ref[...] = hn.astype(jnp.bfloat16)

    @pl.when(pl.program_id(0) == 0)
    def _init():
        st_ref[...] = jnp.zeros_like(st_ref)

    su2 = jnp.sum(hn, axis=0, keepdims=True)
    sq2 = jnp.sum(hn * hn, axis=0, keepdims=True)
    pad = jnp.zeros((6, D), jnp.float32)
    st_ref[...] = st_ref[...] + jnp.concatenate([su2, sq2, pad], axis=0)


def _mlp2_body(h_ref, st2_ref, g_ref, be_ref, w_ref, b_ref, g3_ref, be3_ref,
               o_ref, sf_ref, st_ref):
    st2 = st2_ref[...]
    s, t = _affine_rows(st2[0:1, :], st2[1:2, :], g_ref[...], be_ref[...])
    h = h_ref[...].astype(jnp.float32)
    a = jnp.maximum(h * s + t, 0.0)
    hn = jnp.dot(a.astype(jnp.bfloat16), w_ref[...].astype(jnp.bfloat16),
                 preferred_element_type=jnp.float32) + b_ref[...]
    o_ref[...] = _pack_rows(hn)

    @pl.when(pl.program_id(0) == 0)
    def _init():
        st_ref[...] = jnp.zeros_like(st_ref)

    su2 = jnp.sum(hn, axis=0, keepdims=True)
    sq2 = jnp.sum(hn * hn, axis=0, keepdims=True)
    pad = jnp.zeros((6, D), jnp.float32)
    st_ref[...] = st_ref[...] + jnp.concatenate([su2, sq2, pad], axis=0)

    @pl.when(pl.program_id(0) == E // _BE - 1)
    def _final():
        s3, t3 = _affine_rows(st_ref[0:1, :], st_ref[1:2, :],
                              g3_ref[...], be3_ref[...])
        pad6 = jnp.zeros((6, D), jnp.float32)
        sf_ref[...] = jnp.concatenate([s3, t3, pad6], axis=0)


def _add_body(p_ref, o_ref):
    o_ref[...] = p_ref[0] + p_ref[1]


_BE = 4000
_BN0 = 2000
_BA = 2000


def _y_call(x, w1, b1r):
    return pl.pallas_call(
        _y_body,
        grid=(N // _BN0,),
        in_specs=[
            pl.BlockSpec((_BN0, D), lambda i: (i, 0)),
            pl.BlockSpec((2 * D, D), lambda i: (0, 0)),
            pl.BlockSpec((1, D), lambda i: (0, 0)),
        ],
        out_specs=[
            pl.BlockSpec((_BN0, D), lambda i: (i, 0)),
            pl.BlockSpec((_BN0, D), lambda i: (i, 0)),
        ],
        out_shape=[
            jax.ShapeDtypeStruct((N, D), jnp.float32),
            jax.ShapeDtypeStruct((N, D), jnp.float32),
        ],
    )(x, w1, b1r)


def _mlp1_call(h, stp, g, be, w, b):
    return pl.pallas_call(
        _mlp1_body,
        grid=(E // _BE,),
        in_specs=[
            pl.BlockSpec((_BE, DW), lambda i: (i, 0)),
            pl.BlockSpec((NW, 2 * D), lambda i: (0, 0)),
            pl.BlockSpec((1, D), lambda i: (0, 0)),
            pl.BlockSpec((1, D), lambda i: (0, 0)),
            pl.BlockSpec((D, D), lambda i: (0, 0)),
            pl.BlockSpec((1, D), lambda i: (0, 0)),
        ],
        out_specs=[
            pl.BlockSpec((_BE, D), lambda i: (i, 0)),
            pl.BlockSpec((8, D), lambda i: (0, 0)),
        ],
        out_shape=[
            jax.ShapeDtypeStruct((E, D), jnp.bfloat16),
            jax.ShapeDtypeStruct((8, D), jnp.float32),
        ],
    )(h, stp, g, be, w, b)


def _mlp2_call(h, st2, g, be, w, b, g3, be3):
    return pl.pallas_call(
        _mlp2_body,
        grid=(E // _BE,),
        in_specs=[
            pl.BlockSpec((_BE, D), lambda i: (i, 0)),
            pl.BlockSpec((8, D), lambda i: (0, 0)),
            pl.BlockSpec((1, D), lambda i: (0, 0)),
            pl.BlockSpec((1, D), lambda i: (0, 0)),
            pl.BlockSpec((D, D), lambda i: (0, 0)),
            pl.BlockSpec((1, D), lambda i: (0, 0)),
            pl.BlockSpec((1, D), lambda i: (0, 0)),
            pl.BlockSpec((1, D), lambda i: (0, 0)),
        ],
        out_specs=[
            pl.BlockSpec((_BE, DW), lambda i: (i, 0)),
            pl.BlockSpec((8, D), lambda i: (0, 0)),
            pl.BlockSpec((8, D), lambda i: (0, 0)),
        ],
        out_shape=[
            jax.ShapeDtypeStruct((E, DW), jnp.int32),
            jax.ShapeDtypeStruct((8, D), jnp.float32),
            jax.ShapeDtypeStruct((8, D), jnp.float32),
        ],
    )(h, st2, g, be, w, b, g3, be3)


def _add_call(parts):
    return pl.pallas_call(
        _add_body,
        grid=(N // _BA,),
        in_specs=[pl.BlockSpec((2, _BA, D), lambda i: (0, i, 0))],
        out_specs=pl.BlockSpec((_BA, D), lambda i: (i, 0)),
        out_shape=jax.ShapeDtypeStruct((N, D), jnp.float32),
    )(parts)


def _gather_body(y1_hbm, y2_hbm, ei_hbm, h1_hbm, st_hbm,
                 idx_d, idx_s, rows_d0, rows_d1, rows_s0, rows_s1,
                 hbuf0, hbuf1, statbuf,
                 semd0, semd1, sems0, sems1, semw0, semw1):
    cid = lax.axis_index("c")
    sid = lax.axis_index("s")
    wid = sid * NC + cid
    base = wid * EPW
    rows_d = (rows_d0, rows_d1)
    rows_s = (rows_s0, rows_s1)
    hbuf = (hbuf0, hbuf1)
    semd = (semd0, semd1)
    sems = (sems0, sems1)
    semw = (semw0, semw1)

    pltpu.sync_copy(ei_hbm.at[pl.ds(E + base, EPW)], idx_d)
    pltpu.sync_copy(ei_hbm.at[pl.ds(base, EPW)], idx_s)
    for j in range(16):
        statbuf[pl.ds(j * 16, 16)] = jnp.zeros((16,), jnp.float32)

    def start_gather(b, c):
        off = c * KG
        pltpu.async_copy(y1_hbm.at[idx_d.at[pl.ds(off, KG)]], rows_d[b], semd[b])
        pltpu.async_copy(y2_hbm.at[idx_s.at[pl.ds(off, KG)]], rows_s[b], sems[b])

    start_gather(0, 0)
    start_gather(1, 1)

    def chunk_body(g, _):
        for b in range(2):
            c = 2 * g + b

            @pl.when(c < NCG)
            def _do():
                off = c * KG
                pltpu.make_async_copy(y1_hbm.at[idx_d.at[pl.ds(off, KG)]],
                                      rows_d[b], semd[b]).wait()
                pltpu.make_async_copy(y2_hbm.at[idx_s.at[pl.ds(off, KG)]],
                                      rows_s[b], sems[b]).wait()

                @pl.when(c >= 2)
                def _drain_write():
                    pltpu.make_async_copy(
                        hbuf[b], h1_hbm.at[pl.ds(base + (c - 2) * KG, KG)],
                        semw[b]).wait()

                def row_body(r, a):
                    bc = jax.lax.bitcast_convert_type
                    hs = []
                    sums = []
                    sqs = []
                    for j in range(8):
                        dv = rows_d[b][r, pl.ds(j * 16, 16)]
                        sv = rows_s[b][r, pl.ds(j * 16, 16)]
                        h = dv + sv
                        hs.append(h)
                        sums.append(a[j] + h)
                        sqs.append(a[8 + j] + h * h)
                    for j in range(4):
                        ulo = bc(hs[j], jnp.int32)
                        uhi = bc(hs[4 + j], jnp.int32)
                        hbuf[b][r, pl.ds(j * 16, 16)] = (
                            jax.lax.shift_right_logical(ulo, 16)
                            | (uhi & _HIMASK))
                    return tuple(sums + sqs)

                acc0 = tuple(jnp.zeros((16,), jnp.float32) for _ in range(16))
                acc = lax.fori_loop(0, KG, row_body, acc0)
                for j in range(16):
                    statbuf[pl.ds(j * 16, 16)] = (
                        statbuf[pl.ds(j * 16, 16)] + acc[j])

                pltpu.async_copy(hbuf[b], h1_hbm.at[pl.ds(base + c * KG, KG)],
                                 semw[b])

                @pl.when(c + 2 < NCG)
                def _next():
                    start_gather(b, c + 2)

        return 0

    lax.fori_loop(0, (NCG + 1) // 2, chunk_body, 0)
    pltpu.make_async_copy(hbuf[0], h1_hbm.at[pl.ds(base + (NCG - 1) * KG, KG)],
                          semw[0]).wait()
    pltpu.make_async_copy(hbuf[1], h1_hbm.at[pl.ds(base + (NCG - 2) * KG, KG)],
                          semw[1]).wait()
    pltpu.sync_copy(statbuf, st_hbm.at[wid])


def _gather_call(y1, y2, eflat):
    mesh = plsc.VectorSubcoreMesh(core_axis_name="c", subcore_axis_name="s")
    f = functools.partial(
        pl.kernel,
        mesh=mesh,
        out_type=[
            jax.ShapeDtypeStruct((E, DW), jnp.int32),
            jax.ShapeDtypeStruct((NW, 2 * D), jnp.float32),
        ],
        scratch_types=[
            pltpu.VMEM((EPW,), jnp.int32),
            pltpu.VMEM((EPW,), jnp.int32),
            pltpu.VMEM((KG, D), jnp.float32),
            pltpu.VMEM((KG, D), jnp.float32),
            pltpu.VMEM((KG, D), jnp.float32),
            pltpu.VMEM((KG, D), jnp.float32),
            pltpu.VMEM((KG, DW), jnp.int32),
            pltpu.VMEM((KG, DW), jnp.int32),
            pltpu.VMEM((2 * D,), jnp.float32),
            pltpu.SemaphoreType.DMA,
            pltpu.SemaphoreType.DMA,
            pltpu.SemaphoreType.DMA,
            pltpu.SemaphoreType.DMA,
            pltpu.SemaphoreType.DMA,
            pltpu.SemaphoreType.DMA,
        ],
    )(_gather_body)
    return f(y1, y2, eflat)


def _scatter_body(h3_hbm, ei_hbm, sf_hbm, z_hbm, out_hbm,
                  rbuf0, rbuf1, wbuf0, wbuf1,
                  ib00, ib01, ib10, ib11, stbuf, acc_shared,
                  semr0, semr1, semw0, semw1,
                  semi00, semi01, semi10, semi11):
    cid = lax.axis_index("c")
    sid = lax.axis_index("s")
    wid = sid * NC + cid
    ebase = wid * EPW
    rbuf = (rbuf0, rbuf1)
    wbuf = (wbuf0, wbuf1)
    ibuf = ((ib00, ib01), (ib10, ib11))
    semr = (semr0, semr1)
    semw = (semw0, semw1)
    semi = ((semi00, semi01), (semi10, semi11))

    pltpu.sync_copy(sf_hbm, stbuf)
    pltpu.sync_copy(z_hbm.at[pl.ds(sid * NPT, NPT)],
                    acc_shared.at[pl.ds(sid * NPT, NPT)])

    @pl.when(sid == NS - 1)
    def _zero_tail():
        pltpu.sync_copy(z_hbm.at[pl.ds(NS * NPT, NTAIL)],
                        acc_shared.at[pl.ds(NS * NPT, NTAIL)])

    svs = [stbuf[0, pl.ds(j * 16, 16)] for j in range(8)]
    tvs = [stbuf[1, pl.ds(j * 16, 16)] for j in range(8)]

    def start_read(b, il, c):
        off = ebase + c * KS
        pltpu.async_copy(h3_hbm.at[pl.ds(off, KS)], rbuf[b], semr[b])
        pltpu.async_copy(ei_hbm.at[pl.ds(E + off, KS)], ibuf[b][il],
                         semi[b][il])

    start_read(0, 0, 0)
    start_read(1, 0, 1)
    plsc.subcore_barrier()

    def chunk_body(g, _):
        for q in range(4):
            c = 4 * g + q
            b = q % 2
            il = q // 2

            @pl.when(c < NCS)
            def _do():
                pltpu.make_async_copy(h3_hbm.at[pl.ds(0, KS)], rbuf[b],
                                      semr[b]).wait()
                pltpu.make_async_copy(ei_hbm.at[pl.ds(0, KS)], ibuf[b][il],
                                      semi[b][il]).wait()

                @pl.when(c >= 2)
                def _drain_add():
                    pltpu.make_async_copy(wbuf[b],
                                          acc_shared.at[ibuf[b][1 - il]],
                                          semw[b]).wait()

                def row_body(r, rr):
                    bc = jax.lax.bitcast_convert_type
                    for j in range(4):
                        v = rbuf[b][r, pl.ds(j * 16, 16)]
                        lo = bc(jax.lax.shift_left(v, 16), jnp.float32)
                        hi = bc(v & _HIMASK, jnp.float32)
                        wbuf[b][r, pl.ds(j * 16, 16)] = jnp.maximum(
                            lo * svs[j] + tvs[j], 0.0)
                        wbuf[b][r, pl.ds((4 + j) * 16, 16)] = jnp.maximum(
                            hi * svs[4 + j] + tvs[4 + j], 0.0)
                    return rr

                lax.fori_loop(0, KS, row_body, 0)
                pltpu.async_copy(wbuf[b], acc_shared.at[ibuf[b][il]],
                                 semw[b], add=True)

                @pl.when(c + 2 < NCS)
                def _next():
                    start_read(b, 1 - il, c + 2)

        return 0

    lax.fori_loop(0, (NCS + 3) // 4, chunk_body, 0)
    pltpu.make_async_copy(wbuf[0], acc_shared.at[ibuf[0][0]], semw[0]).wait()
    pltpu.make_async_copy(wbuf[1], acc_shared.at[ibuf[1][1]], semw[1]).wait()
    plsc.subcore_barrier()
    pltpu.sync_copy(acc_shared.at[pl.ds(sid * NPT, NPT)],
                    out_hbm.at[cid, pl.ds(sid * NPT, NPT)])

    @pl.when(sid == NS - 1)
    def _dump_tail():
        pltpu.sync_copy(acc_shared.at[pl.ds(NS * NPT, NTAIL)],
                        out_hbm.at[cid, pl.ds(NS * NPT, NTAIL)])


def _scatter_call(h3, eflat, sf, zeros_nd):
    mesh = plsc.VectorSubcoreMesh(core_axis_name="c", subcore_axis_name="s")
    f = functools.partial(
        pl.kernel,
        mesh=mesh,
        out_type=jax.ShapeDtypeStruct((NC, N, D), jnp.float32),
        scratch_types=[
            pltpu.VMEM((KS, DW), jnp.int32),
            pltpu.VMEM((KS, DW), jnp.int32),
            pltpu.VMEM((KS, D), jnp.float32),
            pltpu.VMEM((KS, D), jnp.float32),
            pltpu.VMEM((KS,), jnp.int32),
            pltpu.VMEM((KS,), jnp.int32),
            pltpu.VMEM((KS,), jnp.int32),
            pltpu.VMEM((KS,), jnp.int32),
            pltpu.VMEM((8, D), jnp.float32),
            pltpu.VMEM_SHARED((N, D), jnp.float32),
            pltpu.SemaphoreType.DMA,
            pltpu.SemaphoreType.DMA,
            pltpu.SemaphoreType.DMA,
            pltpu.SemaphoreType.DMA,
            pltpu.SemaphoreType.DMA,
            pltpu.SemaphoreType.DMA,
            pltpu.SemaphoreType.DMA,
            pltpu.SemaphoreType.DMA,
        ],
    )(_scatter_body)
    return f(h3, eflat, sf, zeros_nd)


def kernel(X, edge_index, W1, b1, g1, be1, W2, b2, g2, be2, W3, b3, g3, be3):
    eflat = edge_index.astype(jnp.int32).reshape(2 * E)

    y1, y2 = _y_call(X, W1, b1.reshape(1, D))
    h1, st1p = _gather_call(y1, y2, eflat)
    h2, st2 = _mlp1_call(h1, st1p, g1.reshape(1, D), be1.reshape(1, D),
                         W2, b2.reshape(1, D))
    h3, sf, _st3 = _mlp2_call(h2, st2, g2.reshape(1, D), be2.reshape(1, D),
                              W3, b3.reshape(1, D),
                              g3.reshape(1, D), be3.reshape(1, D))
    parts = _scatter_call(h3, eflat, sf, jnp.zeros((N, D), jnp.float32))
    return _add_call(parts)


# speedup vs baseline: 1.0523x; 1.0469x over previous
import functools

import jax
import jax.numpy as jnp
from jax import lax
from jax.experimental import pallas as pl
from jax.experimental.pallas import tpu as pltpu
from jax.experimental.pallas import tpu_sc as plsc

N = 10000
E = 320000
D = 128
DW = D // 2
EPS = 1e-5

NC = 2
NS = 16
NW = NC * NS
EPW = E // NW
KG = 80
NCG = EPW // KG
KS = 80
NCS = EPW // KS
NPT = 624
NTAIL = N - NS * NPT

_HIMASK = -65536


def _pack_rows(y):
    u = jax.lax.bitcast_convert_type(y, jnp.int32)
    r = u + 0x7FFF + (jax.lax.shift_right_logical(u, 16) & 1)
    top = jax.lax.shift_right_logical(r, 16)
    return top[:, :DW] | jax.lax.shift_left(top[:, DW:], 16)


def _unpack_rows(hw):
    lo = jax.lax.bitcast_convert_type(jax.lax.shift_left(hw, 16), jnp.float32)
    hi = jax.lax.bitcast_convert_type(hw & _HIMASK, jnp.float32)
    return jnp.concatenate([lo, hi], axis=1)


def _y_body(x_ref, w1_ref, b1_ref, y1_ref, y2_ref):
    x = x_ref[...]
    wb = w1_ref[128:256, :]
    wd = w1_ref[0:128, :] - wb
    y1_ref[...] = jnp.dot(x, wd, preferred_element_type=jnp.float32) + b1_ref[...]
    y2_ref[...] = jnp.dot(x, wb, preferred_element_type=jnp.float32)


def _affine_rows(su, sq, g, be):
    m = su * (1.0 / E)
    v = sq * (1.0 / E) - m * m
    s = g * lax.rsqrt(v + EPS)
    t = be - m * s
    return s, t


def _mlp1_body(h_ref, stp_ref, g_ref, be_ref, w_ref, b_ref, o_ref, st_ref):
    stp = stp_ref[...]
    su = jnp.sum(stp[:, :D], axis=0, keepdims=True)
    sq = jnp.sum(stp[:, D:], axis=0, keepdims=True)
    s, t = _affine_rows(su, sq, g_ref[...], be_ref[...])
    h = _unpack_rows(h_ref[...])
    a = jnp.maximum(h * s + t, 0.0)
    hn = jnp.dot(a.astype(jnp.bfloat16), w_ref[...].astype(jnp.bfloat16),
                 preferred_element_type=jnp.float32) + b_ref[...]
    hnb = hn.astype(jnp.bfloat16)
    o_ref[...] = hnb

    @pl.when(pl.program_id(0) == 0)
    def _init():
        st_ref[...] = jnp.zeros_like(st_ref)

    ones = jnp.ones((1, _BE), jnp.bfloat16)
    hq = (hn * hn).astype(jnp.bfloat16)
    su2 = jnp.dot(ones, hnb, preferred_element_type=jnp.float32)
    sq2 = jnp.dot(ones, hq, preferred_element_type=jnp.float32)
    pad = jnp.zeros((6, D), jnp.float32)
    st_ref[...] = st_ref[...] + jnp.concatenate([su2, sq2, pad], axis=0)


def _mlp2_body(h_ref, st2_ref, g_ref, be_ref, w_ref, b_ref, g3_ref, be3_ref,
               o_ref, sf_ref, st_ref):
    st2 = st2_ref[...]
    s, t = _affine_rows(st2[0:1, :], st2[1:2, :], g_ref[...], be_ref[...])
    h = h_ref[...].astype(jnp.float32)
    a = jnp.maximum(h * s + t, 0.0)
    hn = jnp.dot(a.astype(jnp.bfloat16), w_ref[...].astype(jnp.bfloat16),
                 preferred_element_type=jnp.float32) + b_ref[...]
    o_ref[...] = _pack_rows(hn)

    @pl.when(pl.program_id(0) == 0)
    def _init():
        st_ref[...] = jnp.zeros_like(st_ref)

    ones = jnp.ones((1, _BE), jnp.bfloat16)
    hnb = hn.astype(jnp.bfloat16)
    hq = (hn * hn).astype(jnp.bfloat16)
    su2 = jnp.dot(ones, hnb, preferred_element_type=jnp.float32)
    sq2 = jnp.dot(ones, hq, preferred_element_type=jnp.float32)
    pad = jnp.zeros((6, D), jnp.float32)
    st_ref[...] = st_ref[...] + jnp.concatenate([su2, sq2, pad], axis=0)

    @pl.when(pl.program_id(0) == E // _BE - 1)
    def _final():
        s3, t3 = _affine_rows(st_ref[0:1, :], st_ref[1:2, :],
                              g3_ref[...], be3_ref[...])
        pad6 = jnp.zeros((6, D), jnp.float32)
        sf_ref[...] = jnp.concatenate([s3, t3, pad6], axis=0)


def _add_body(p_ref, o_ref):
    o_ref[...] = p_ref[0] + p_ref[1]


_BE = 8000
_BN0 = 2000
_BA = 2000


def _y_call(x, w1, b1r):
    return pl.pallas_call(
        _y_body,
        grid=(N // _BN0,),
        in_specs=[
            pl.BlockSpec((_BN0, D), lambda i: (i, 0)),
            pl.BlockSpec((2 * D, D), lambda i: (0, 0)),
            pl.BlockSpec((1, D), lambda i: (0, 0)),
        ],
        out_specs=[
            pl.BlockSpec((_BN0, D), lambda i: (i, 0)),
            pl.BlockSpec((_BN0, D), lambda i: (i, 0)),
        ],
        out_shape=[
            jax.ShapeDtypeStruct((N, D), jnp.float32),
            jax.ShapeDtypeStruct((N, D), jnp.float32),
        ],
    )(x, w1, b1r)


def _mlp1_call(h, stp, g, be, w, b):
    return pl.pallas_call(
        _mlp1_body,
        grid=(E // _BE,),
        in_specs=[
            pl.BlockSpec((_BE, DW), lambda i: (i, 0)),
            pl.BlockSpec((NW, 2 * D), lambda i: (0, 0)),
            pl.BlockSpec((1, D), lambda i: (0, 0)),
            pl.BlockSpec((1, D), lambda i: (0, 0)),
            pl.BlockSpec((D, D), lambda i: (0, 0)),
            pl.BlockSpec((1, D), lambda i: (0, 0)),
        ],
        out_specs=[
            pl.BlockSpec((_BE, D), lambda i: (i, 0)),
            pl.BlockSpec((8, D), lambda i: (0, 0)),
        ],
        out_shape=[
            jax.ShapeDtypeStruct((E, D), jnp.bfloat16),
            jax.ShapeDtypeStruct((8, D), jnp.float32),
        ],
    )(h, stp, g, be, w, b)


def _mlp2_call(h, st2, g, be, w, b, g3, be3):
    return pl.pallas_call(
        _mlp2_body,
        grid=(E // _BE,),
        in_specs=[
            pl.BlockSpec((_BE, D), lambda i: (i, 0)),
            pl.BlockSpec((8, D), lambda i: (0, 0)),
            pl.BlockSpec((1, D), lambda i: (0, 0)),
            pl.BlockSpec((1, D), lambda i: (0, 0)),
            pl.BlockSpec((D, D), lambda i: (0, 0)),
            pl.BlockSpec((1, D), lambda i: (0, 0)),
            pl.BlockSpec((1, D), lambda i: (0, 0)),
            pl.BlockSpec((1, D), lambda i: (0, 0)),
        ],
        out_specs=[
            pl.BlockSpec((_BE, DW), lambda i: (i, 0)),
            pl.BlockSpec((8, D), lambda i: (0, 0)),
            pl.BlockSpec((8, D), lambda i: (0, 0)),
        ],
        out_shape=[
            jax.ShapeDtypeStruct((E, DW), jnp.int32),
            jax.ShapeDtypeStruct((8, D), jnp.float32),
            jax.ShapeDtypeStruct((8, D), jnp.float32),
        ],
    )(h, st2, g, be, w, b, g3, be3)


def _add_call(parts):
    return pl.pallas_call(
        _add_body,
        grid=(N // _BA,),
        in_specs=[pl.BlockSpec((2, _BA, D), lambda i: (0, i, 0))],
        out_specs=pl.BlockSpec((_BA, D), lambda i: (i, 0)),
        out_shape=jax.ShapeDtypeStruct((N, D), jnp.float32),
    )(parts)


def _gather_body(y1_hbm, y2_hbm, ei_hbm, h1_hbm, st_hbm,
                 idx_d, idx_s, rows_d0, rows_d1, rows_s0, rows_s1,
                 hbuf0, hbuf1, statbuf,
                 semd0, semd1, sems0, sems1, semw0, semw1):
    cid = lax.axis_index("c")
    sid = lax.axis_index("s")
    wid = sid * NC + cid
    base = wid * EPW
    rows_d = (rows_d0, rows_d1)
    rows_s = (rows_s0, rows_s1)
    hbuf = (hbuf0, hbuf1)
    semd = (semd0, semd1)
    sems = (sems0, sems1)
    semw = (semw0, semw1)

    pltpu.sync_copy(ei_hbm.at[pl.ds(E + base, EPW)], idx_d)
    pltpu.sync_copy(ei_hbm.at[pl.ds(base, EPW)], idx_s)
    for j in range(16):
        statbuf[pl.ds(j * 16, 16)] = jnp.zeros((16,), jnp.float32)

    def start_gather(b, c):
        off = c * KG
        pltpu.async_copy(y1_hbm.at[idx_d.at[pl.ds(off, KG)]], rows_d[b], semd[b])
        pltpu.async_copy(y2_hbm.at[idx_s.at[pl.ds(off, KG)]], rows_s[b], sems[b])

    start_gather(0, 0)
    start_gather(1, 1)

    def chunk_body(g, _):
        for b in range(2):
            c = 2 * g + b

            @pl.when(c < NCG)
            def _do():
                off = c * KG
                pltpu.make_async_copy(y1_hbm.at[idx_d.at[pl.ds(off, KG)]],
                                      rows_d[b], semd[b]).wait()
                pltpu.make_async_copy(y2_hbm.at[idx_s.at[pl.ds(off, KG)]],
                                      rows_s[b], sems[b]).wait()

                @pl.when(c >= 2)
                def _drain_write():
                    pltpu.make_async_copy(
                        hbuf[b], h1_hbm.at[pl.ds(base + (c - 2) * KG, KG)],
                        semw[b]).wait()

                def row_body(r, a):
                    bc = jax.lax.bitcast_convert_type
                    hs = []
                    sums = []
                    sqs = []
                    for j in range(8):
                        dv = rows_d[b][r, pl.ds(j * 16, 16)]
                        sv = rows_s[b][r, pl.ds(j * 16, 16)]
                        h = dv + sv
                        hs.append(h)
                        sums.append(a[j] + h)
                        sqs.append(a[8 + j] + h * h)
                    for j in range(4):
                        ulo = bc(hs[j], jnp.int32)
                        uhi = bc(hs[4 + j], jnp.int32)
                        hbuf[b][r, pl.ds(j * 16, 16)] = (
                            jax.lax.shift_right_logical(ulo, 16)
                            | (uhi & _HIMASK))
                    return tuple(sums + sqs)

                acc0 = tuple(jnp.zeros((16,), jnp.float32) for _ in range(16))
                acc = lax.fori_loop(0, KG, row_body, acc0)
                for j in range(16):
                    statbuf[pl.ds(j * 16, 16)] = (
                        statbuf[pl.ds(j * 16, 16)] + acc[j])

                pltpu.async_copy(hbuf[b], h1_hbm.at[pl.ds(base + c * KG, KG)],
                                 semw[b])

                @pl.when(c + 2 < NCG)
                def _next():
                    start_gather(b, c + 2)

        return 0

    lax.fori_loop(0, (NCG + 1) // 2, chunk_body, 0)
    pltpu.make_async_copy(hbuf[0], h1_hbm.at[pl.ds(base + (NCG - 1) * KG, KG)],
                          semw[0]).wait()
    pltpu.make_async_copy(hbuf[1], h1_hbm.at[pl.ds(base + (NCG - 2) * KG, KG)],
                          semw[1]).wait()
    pltpu.sync_copy(statbuf, st_hbm.at[wid])


def _gather_call(y1, y2, eflat):
    mesh = plsc.VectorSubcoreMesh(core_axis_name="c", subcore_axis_name="s")
    f = functools.partial(
        pl.kernel,
        mesh=mesh,
        out_type=[
            jax.ShapeDtypeStruct((E, DW), jnp.int32),
            jax.ShapeDtypeStruct((NW, 2 * D), jnp.float32),
        ],
        scratch_types=[
            pltpu.VMEM((EPW,), jnp.int32),
            pltpu.VMEM((EPW,), jnp.int32),
            pltpu.VMEM((KG, D), jnp.float32),
            pltpu.VMEM((KG, D), jnp.float32),
            pltpu.VMEM((KG, D), jnp.float32),
            pltpu.VMEM((KG, D), jnp.float32),
            pltpu.VMEM((KG, DW), jnp.int32),
            pltpu.VMEM((KG, DW), jnp.int32),
            pltpu.VMEM((2 * D,), jnp.float32),
            pltpu.SemaphoreType.DMA,
            pltpu.SemaphoreType.DMA,
            pltpu.SemaphoreType.DMA,
            pltpu.SemaphoreType.DMA,
            pltpu.SemaphoreType.DMA,
            pltpu.SemaphoreType.DMA,
        ],
    )(_gather_body)
    return f(y1, y2, eflat)


def _scatter_body(h3_hbm, ei_hbm, sf_hbm, z_hbm, out_hbm,
                  rbuf0, rbuf1, wbuf0, wbuf1,
                  ib00, ib01, ib10, ib11, stbuf, acc_shared,
                  semr0, semr1, semw0, semw1,
                  semi00, semi01, semi10, semi11):
    cid = lax.axis_index("c")
    sid = lax.axis_index("s")
    wid = sid * NC + cid
    ebase = wid * EPW
    rbuf = (rbuf0, rbuf1)
    wbuf = (wbuf0, wbuf1)
    ibuf = ((ib00, ib01), (ib10, ib11))
    semr = (semr0, semr1)
    semw = (semw0, semw1)
    semi = ((semi00, semi01), (semi10, semi11))

    pltpu.sync_copy(sf_hbm, stbuf)
    pltpu.sync_copy(z_hbm.at[pl.ds(sid * NPT, NPT)],
                    acc_shared.at[pl.ds(sid * NPT, NPT)])

    @pl.when(sid == NS - 1)
    def _zero_tail():
        pltpu.sync_copy(z_hbm.at[pl.ds(NS * NPT, NTAIL)],
                        acc_shared.at[pl.ds(NS * NPT, NTAIL)])

    svs = [stbuf[0, pl.ds(j * 16, 16)] for j in range(8)]
    tvs = [stbuf[1, pl.ds(j * 16, 16)] for j in range(8)]

    def start_read(b, il, c):
        off = ebase + c * KS
        pltpu.async_copy(h3_hbm.at[pl.ds(off, KS)], rbuf[b], semr[b])
        pltpu.async_copy(ei_hbm.at[pl.ds(E + off, KS)], ibuf[b][il],
                         semi[b][il])

    start_read(0, 0, 0)
    start_read(1, 0, 1)
    plsc.subcore_barrier()

    def chunk_body(g, _):
        for q in range(4):
            c = 4 * g + q
            b = q % 2
            il = q // 2

            @pl.when(c < NCS)
            def _do():
                pltpu.make_async_copy(h3_hbm.at[pl.ds(0, KS)], rbuf[b],
                                      semr[b]).wait()
                pltpu.make_async_copy(ei_hbm.at[pl.ds(0, KS)], ibuf[b][il],
                                      semi[b][il]).wait()

                @pl.when(c >= 2)
                def _drain_add():
                    pltpu.make_async_copy(wbuf[b],
                                          acc_shared.at[ibuf[b][1 - il]],
                                          semw[b]).wait()

                def row_body(r, rr):
                    bc = jax.lax.bitcast_convert_type
                    for j in range(4):
                        v = rbuf[b][r, pl.ds(j * 16, 16)]
                        lo = bc(jax.lax.shift_left(v, 16), jnp.float32)
                        hi = bc(v & _HIMASK, jnp.float32)
                        wbuf[b][r, pl.ds(j * 16, 16)] = jnp.maximum(
                            lo * svs[j] + tvs[j], 0.0)
                        wbuf[b][r, pl.ds((4 + j) * 16, 16)] = jnp.maximum(
                            hi * svs[4 + j] + tvs[4 + j], 0.0)
                    return rr

                lax.fori_loop(0, KS, row_body, 0)
                pltpu.async_copy(wbuf[b], acc_shared.at[ibuf[b][il]],
                                 semw[b], add=True)

                @pl.when(c + 2 < NCS)
                def _next():
                    start_read(b, 1 - il, c + 2)

        return 0

    lax.fori_loop(0, (NCS + 3) // 4, chunk_body, 0)
    pltpu.make_async_copy(wbuf[0], acc_shared.at[ibuf[0][0]], semw[0]).wait()
    pltpu.make_async_copy(wbuf[1], acc_shared.at[ibuf[1][1]], semw[1]).wait()
    plsc.subcore_barrier()
    pltpu.sync_copy(acc_shared.at[pl.ds(sid * NPT, NPT)],
                    out_hbm.at[cid, pl.ds(sid * NPT, NPT)])

    @pl.when(sid == NS - 1)
    def _dump_tail():
        pltpu.sync_copy(acc_shared.at[pl.ds(NS * NPT, NTAIL)],
                        out_hbm.at[cid, pl.ds(NS * NPT, NTAIL)])


def _scatter_call(h3, eflat, sf, zeros_nd):
    mesh = plsc.VectorSubcoreMesh(core_axis_name="c", subcore_axis_name="s")
    f = functools.partial(
        pl.kernel,
        mesh=mesh,
        out_type=jax.ShapeDtypeStruct((NC, N, D), jnp.float32),
        scratch_types=[
            pltpu.VMEM((KS, DW), jnp.int32),
            pltpu.VMEM((KS, DW), jnp.int32),
            pltpu.VMEM((KS, D), jnp.float32),
            pltpu.VMEM((KS, D), jnp.float32),
            pltpu.VMEM((KS,), jnp.int32),
            pltpu.VMEM((KS,), jnp.int32),
            pltpu.VMEM((KS,), jnp.int32),
            pltpu.VMEM((KS,), jnp.int32),
            pltpu.VMEM((8, D), jnp.float32),
            pltpu.VMEM_SHARED((N, D), jnp.float32),
            pltpu.SemaphoreType.DMA,
            pltpu.SemaphoreType.DMA,
            pltpu.SemaphoreType.DMA,
            pltpu.SemaphoreType.DMA,
            pltpu.SemaphoreType.DMA,
            pltpu.SemaphoreType.DMA,
            pltpu.SemaphoreType.DMA,
            pltpu.SemaphoreType.DMA,
        ],
    )(_scatter_body)
    return f(h3, eflat, sf, zeros_nd)


def kernel(X, edge_index, W1, b1, g1, be1, W2, b2, g2, be2, W3, b3, g3, be3):
    eflat = edge_index.astype(jnp.int32).reshape(2 * E)

    y1, y2 = _y_call(X, W1, b1.reshape(1, D))
    h1, st1p = _gather_call(y1, y2, eflat)
    h2, st2 = _mlp1_call(h1, st1p, g1.reshape(1, D), be1.reshape(1, D),
                         W2, b2.reshape(1, D))
    h3, sf, _st3 = _mlp2_call(h2, st2, g2.reshape(1, D), be2.reshape(1, D),
                              W3, b3.reshape(1, D),
                              g3.reshape(1, D), be3.reshape(1, D))
    parts = _scatter_call(h3, eflat, sf, jnp.zeros((N, D), jnp.float32))
    return _add_call(parts)


# speedup vs baseline: 1.0625x; 1.0097x over previous
import functools

import jax
import jax.numpy as jnp
from jax import lax
from jax.experimental import pallas as pl
from jax.experimental.pallas import tpu as pltpu
from jax.experimental.pallas import tpu_sc as plsc

N = 10000
E = 320000
D = 128
DW = D // 2
EPS = 1e-5

NC = 2
NS = 16
NW = NC * NS
EPW = E // NW
KG = 80
NCG = EPW // KG
KS = 80
NCS = EPW // KS
NPT = 624
NTAIL = N - NS * NPT

_HIMASK = -65536


def _pack_rows(y):
    u = jax.lax.bitcast_convert_type(y, jnp.int32)
    r = u + 0x7FFF + (jax.lax.shift_right_logical(u, 16) & 1)
    top = jax.lax.shift_right_logical(r, 16)
    return top[:, :DW] | jax.lax.shift_left(top[:, DW:], 16)


def _unpack_rows(hw):
    lo = jax.lax.bitcast_convert_type(jax.lax.shift_left(hw, 16), jnp.float32)
    hi = jax.lax.bitcast_convert_type(hw & _HIMASK, jnp.float32)
    return jnp.concatenate([lo, hi], axis=1)


def _y_body(x_ref, w1_ref, b1_ref, y1_ref, y2_ref):
    x = x_ref[...]
    wb = w1_ref[128:256, :]
    wd = w1_ref[0:128, :] - wb
    y1_ref[...] = jnp.dot(x, wd, preferred_element_type=jnp.float32) + b1_ref[...]
    y2_ref[...] = jnp.dot(x, wb, preferred_element_type=jnp.float32)


def _affine_rows(su, sq, g, be):
    m = su * (1.0 / E)
    v = sq * (1.0 / E) - m * m
    s = g * lax.rsqrt(v + EPS)
    t = be - m * s
    return s, t


def _mlp1_body(h_ref, stp_ref, g_ref, be_ref, w_ref, b_ref, o_ref, st_ref):
    stp = stp_ref[...]
    su = jnp.sum(stp[:, :D], axis=0, keepdims=True)
    sq = jnp.sum(stp[:, D:], axis=0, keepdims=True)
    s, t = _affine_rows(su, sq, g_ref[...], be_ref[...])
    h = _unpack_rows(h_ref[...])
    a = jnp.maximum(h * s + t, 0.0)
    hn = jnp.dot(a.astype(jnp.bfloat16), w_ref[...].astype(jnp.bfloat16),
                 preferred_element_type=jnp.float32) + b_ref[...]
    hnb = hn.astype(jnp.bfloat16)
    o_ref[...] = hnb

    @pl.when(pl.program_id(0) == 0)
    def _init():
        st_ref[...] = jnp.zeros_like(st_ref)

    ones = jnp.ones((1, _BE), jnp.bfloat16)
    hq = (hn * hn).astype(jnp.bfloat16)
    su2 = jnp.dot(ones, hnb, preferred_element_type=jnp.float32)
    sq2 = jnp.dot(ones, hq, preferred_element_type=jnp.float32)
    pad = jnp.zeros((6, D), jnp.float32)
    st_ref[...] = st_ref[...] + jnp.concatenate([su2, sq2, pad], axis=0)


def _mlp2_body(h_ref, st2_ref, g_ref, be_ref, w_ref, b_ref, g3_ref, be3_ref,
               o_ref, sf_ref, st_ref):
    st2 = st2_ref[...]
    s, t = _affine_rows(st2[0:1, :], st2[1:2, :], g_ref[...], be_ref[...])
    h = h_ref[...].astype(jnp.float32)
    a = jnp.maximum(h * s + t, 0.0)
    hn = jnp.dot(a.astype(jnp.bfloat16), w_ref[...].astype(jnp.bfloat16),
                 preferred_element_type=jnp.float32) + b_ref[...]
    o_ref[...] = _pack_rows(hn)

    @pl.when(pl.program_id(0) == 0)
    def _init():
        st_ref[...] = jnp.zeros_like(st_ref)

    ones = jnp.ones((1, _BE), jnp.bfloat16)
    hnb = hn.astype(jnp.bfloat16)
    hq = (hn * hn).astype(jnp.bfloat16)
    su2 = jnp.dot(ones, hnb, preferred_element_type=jnp.float32)
    sq2 = jnp.dot(ones, hq, preferred_element_type=jnp.float32)
    pad = jnp.zeros((6, D), jnp.float32)
    st_ref[...] = st_ref[...] + jnp.concatenate([su2, sq2, pad], axis=0)

    @pl.when(pl.program_id(0) == E // _BE - 1)
    def _final():
        s3, t3 = _affine_rows(st_ref[0:1, :], st_ref[1:2, :],
                              g3_ref[...], be3_ref[...])
        pad6 = jnp.zeros((6, D), jnp.float32)
        sf_ref[...] = jnp.concatenate([s3, t3, pad6], axis=0)


def _add_body(p_ref, o_ref):
    o_ref[...] = p_ref[0] + p_ref[1]


_BE = 16000
_BN0 = 2000
_BA = 2000


def _y_call(x, w1, b1r):
    return pl.pallas_call(
        _y_body,
        grid=(N // _BN0,),
        in_specs=[
            pl.BlockSpec((_BN0, D), lambda i: (i, 0)),
            pl.BlockSpec((2 * D, D), lambda i: (0, 0)),
            pl.BlockSpec((1, D), lambda i: (0, 0)),
        ],
        out_specs=[
            pl.BlockSpec((_BN0, D), lambda i: (i, 0)),
            pl.BlockSpec((_BN0, D), lambda i: (i, 0)),
        ],
        out_shape=[
            jax.ShapeDtypeStruct((N, D), jnp.float32),
            jax.ShapeDtypeStruct((N, D), jnp.float32),
        ],
    )(x, w1, b1r)


def _mlp1_call(h, stp, g, be, w, b):
    return pl.pallas_call(
        _mlp1_body,
        grid=(E // _BE,),
        in_specs=[
            pl.BlockSpec((_BE, DW), lambda i: (i, 0)),
            pl.BlockSpec((NW, 2 * D), lambda i: (0, 0)),
            pl.BlockSpec((1, D), lambda i: (0, 0)),
            pl.BlockSpec((1, D), lambda i: (0, 0)),
            pl.BlockSpec((D, D), lambda i: (0, 0)),
            pl.BlockSpec((1, D), lambda i: (0, 0)),
        ],
        out_specs=[
            pl.BlockSpec((_BE, D), lambda i: (i, 0)),
            pl.BlockSpec((8, D), lambda i: (0, 0)),
        ],
        out_shape=[
            jax.ShapeDtypeStruct((E, D), jnp.bfloat16),
            jax.ShapeDtypeStruct((8, D), jnp.float32),
        ],
    )(h, stp, g, be, w, b)


def _mlp2_call(h, st2, g, be, w, b, g3, be3):
    return pl.pallas_call(
        _mlp2_body,
        grid=(E // _BE,),
        in_specs=[
            pl.BlockSpec((_BE, D), lambda i: (i, 0)),
            pl.BlockSpec((8, D), lambda i: (0, 0)),
            pl.BlockSpec((1, D), lambda i: (0, 0)),
            pl.BlockSpec((1, D), lambda i: (0, 0)),
            pl.BlockSpec((D, D), lambda i: (0, 0)),
            pl.BlockSpec((1, D), lambda i: (0, 0)),
            pl.BlockSpec((1, D), lambda i: (0, 0)),
            pl.BlockSpec((1, D), lambda i: (0, 0)),
        ],
        out_specs=[
            pl.BlockSpec((_BE, DW), lambda i: (i, 0)),
            pl.BlockSpec((8, D), lambda i: (0, 0)),
            pl.BlockSpec((8, D), lambda i: (0, 0)),
        ],
        out_shape=[
            jax.ShapeDtypeStruct((E, DW), jnp.int32),
            jax.ShapeDtypeStruct((8, D), jnp.float32),
            jax.ShapeDtypeStruct((8, D), jnp.float32),
        ],
    )(h, st2, g, be, w, b, g3, be3)


def _add_call(parts):
    return pl.pallas_call(
        _add_body,
        grid=(N // _BA,),
        in_specs=[pl.BlockSpec((2, _BA, D), lambda i: (0, i, 0))],
        out_specs=pl.BlockSpec((_BA, D), lambda i: (i, 0)),
        out_shape=jax.ShapeDtypeStruct((N, D), jnp.float32),
    )(parts)


def _gather_body(y1_hbm, y2_hbm, ei_hbm, h1_hbm, st_hbm,
                 idx_d, idx_s, rows_d0, rows_d1, rows_s0, rows_s1,
                 hbuf0, hbuf1, statbuf,
                 semd0, semd1, sems0, sems1, semw0, semw1):
    cid = lax.axis_index("c")
    sid = lax.axis_index("s")
    wid = sid * NC + cid
    base = wid * EPW
    rows_d = (rows_d0, rows_d1)
    rows_s = (rows_s0, rows_s1)
    hbuf = (hbuf0, hbuf1)
    semd = (semd0, semd1)
    sems = (sems0, sems1)
    semw = (semw0, semw1)

    pltpu.sync_copy(ei_hbm.at[pl.ds(E + base, EPW)], idx_d)
    pltpu.sync_copy(ei_hbm.at[pl.ds(base, EPW)], idx_s)
    for j in range(16):
        statbuf[pl.ds(j * 16, 16)] = jnp.zeros((16,), jnp.float32)

    def start_gather(b, c):
        off = c * KG
        pltpu.async_copy(y1_hbm.at[idx_d.at[pl.ds(off, KG)]], rows_d[b], semd[b])
        pltpu.async_copy(y2_hbm.at[idx_s.at[pl.ds(off, KG)]], rows_s[b], sems[b])

    start_gather(0, 0)
    start_gather(1, 1)

    def chunk_body(g, _):
        for b in range(2):
            c = 2 * g + b

            @pl.when(c < NCG)
            def _do():
                off = c * KG
                pltpu.make_async_copy(y1_hbm.at[idx_d.at[pl.ds(off, KG)]],
                                      rows_d[b], semd[b]).wait()
                pltpu.make_async_copy(y2_hbm.at[idx_s.at[pl.ds(off, KG)]],
                                      rows_s[b], sems[b]).wait()

                @pl.when(c >= 2)
                def _drain_write():
                    pltpu.make_async_copy(
                        hbuf[b], h1_hbm.at[pl.ds(base + (c - 2) * KG, KG)],
                        semw[b]).wait()

                def row_body(r, a):
                    bc = jax.lax.bitcast_convert_type
                    hs = []
                    sums = []
                    sqs = []
                    for j in range(8):
                        dv = rows_d[b][r, pl.ds(j * 16, 16)]
                        sv = rows_s[b][r, pl.ds(j * 16, 16)]
                        h = dv + sv
                        hs.append(h)
                        sums.append(a[j] + h)
                        sqs.append(a[8 + j] + h * h)
                    for j in range(4):
                        ulo = bc(hs[j], jnp.int32)
                        uhi = bc(hs[4 + j], jnp.int32)
                        hbuf[b][r, pl.ds(j * 16, 16)] = (
                            jax.lax.shift_right_logical(ulo, 16)
                            | (uhi & _HIMASK))
                    return tuple(sums + sqs)

                acc0 = tuple(jnp.zeros((16,), jnp.float32) for _ in range(16))
                acc = lax.fori_loop(0, KG, row_body, acc0)
                for j in range(16):
                    statbuf[pl.ds(j * 16, 16)] = (
                        statbuf[pl.ds(j * 16, 16)] + acc[j])

                pltpu.async_copy(hbuf[b], h1_hbm.at[pl.ds(base + c * KG, KG)],
                                 semw[b])

                @pl.when(c + 2 < NCG)
                def _next():
                    start_gather(b, c + 2)

        return 0

    lax.fori_loop(0, (NCG + 1) // 2, chunk_body, 0)
    pltpu.make_async_copy(hbuf[0], h1_hbm.at[pl.ds(base + (NCG - 1) * KG, KG)],
                          semw[0]).wait()
    pltpu.make_async_copy(hbuf[1], h1_hbm.at[pl.ds(base + (NCG - 2) * KG, KG)],
                          semw[1]).wait()
    pltpu.sync_copy(statbuf, st_hbm.at[wid])


def _gather_call(y1, y2, eflat):
    mesh = plsc.VectorSubcoreMesh(core_axis_name="c", subcore_axis_name="s")
    f = functools.partial(
        pl.kernel,
        mesh=mesh,
        out_type=[
            jax.ShapeDtypeStruct((E, DW), jnp.int32),
            jax.ShapeDtypeStruct((NW, 2 * D), jnp.float32),
        ],
        scratch_types=[
            pltpu.VMEM((EPW,), jnp.int32),
            pltpu.VMEM((EPW,), jnp.int32),
            pltpu.VMEM((KG, D), jnp.float32),
            pltpu.VMEM((KG, D), jnp.float32),
            pltpu.VMEM((KG, D), jnp.float32),
            pltpu.VMEM((KG, D), jnp.float32),
            pltpu.VMEM((KG, DW), jnp.int32),
            pltpu.VMEM((KG, DW), jnp.int32),
            pltpu.VMEM((2 * D,), jnp.float32),
            pltpu.SemaphoreType.DMA,
            pltpu.SemaphoreType.DMA,
            pltpu.SemaphoreType.DMA,
            pltpu.SemaphoreType.DMA,
            pltpu.SemaphoreType.DMA,
            pltpu.SemaphoreType.DMA,
        ],
    )(_gather_body)
    return f(y1, y2, eflat)


def _scatter_body(h3_hbm, ei_hbm, sf_hbm, z_hbm, out_hbm,
                  rbuf0, rbuf1, wbuf0, wbuf1,
                  ib00, ib01, ib10, ib11, stbuf, acc_shared,
                  semr0, semr1, semw0, semw1,
                  semi00, semi01, semi10, semi11):
    cid = lax.axis_index("c")
    sid = lax.axis_index("s")
    wid = sid * NC + cid
    ebase = wid * EPW
    rbuf = (rbuf0, rbuf1)
    wbuf = (wbuf0, wbuf1)
    ibuf = ((ib00, ib01), (ib10, ib11))
    semr = (semr0, semr1)
    semw = (semw0, semw1)
    semi = ((semi00, semi01), (semi10, semi11))

    pltpu.sync_copy(sf_hbm, stbuf)
    pltpu.sync_copy(z_hbm.at[pl.ds(sid * NPT, NPT)],
                    acc_shared.at[pl.ds(sid * NPT, NPT)])

    @pl.when(sid == NS - 1)
    def _zero_tail():
        pltpu.sync_copy(z_hbm.at[pl.ds(NS * NPT, NTAIL)],
                        acc_shared.at[pl.ds(NS * NPT, NTAIL)])

    svs = [stbuf[0, pl.ds(j * 16, 16)] for j in range(8)]
    tvs = [stbuf[1, pl.ds(j * 16, 16)] for j in range(8)]

    def start_read(b, il, c):
        off = ebase + c * KS
        pltpu.async_copy(h3_hbm.at[pl.ds(off, KS)], rbuf[b], semr[b])
        pltpu.async_copy(ei_hbm.at[pl.ds(E + off, KS)], ibuf[b][il],
                         semi[b][il])

    start_read(0, 0, 0)
    start_read(1, 0, 1)
    plsc.subcore_barrier()

    def chunk_body(g, _):
        for q in range(4):
            c = 4 * g + q
            b = q % 2
            il = q // 2

            @pl.when(c < NCS)
            def _do():
                pltpu.make_async_copy(h3_hbm.at[pl.ds(0, KS)], rbuf[b],
                                      semr[b]).wait()
                pltpu.make_async_copy(ei_hbm.at[pl.ds(0, KS)], ibuf[b][il],
                                      semi[b][il]).wait()

                @pl.when(c >= 2)
                def _drain_add():
                    pltpu.make_async_copy(wbuf[b],
                                          acc_shared.at[ibuf[b][1 - il]],
                                          semw[b]).wait()

                def row_body(r, rr):
                    bc = jax.lax.bitcast_convert_type
                    for j in range(4):
                        v = rbuf[b][r, pl.ds(j * 16, 16)]
                        lo = bc(jax.lax.shift_left(v, 16), jnp.float32)
                        hi = bc(v & _HIMASK, jnp.float32)
                        wbuf[b][r, pl.ds(j * 16, 16)] = jnp.maximum(
                            lo * svs[j] + tvs[j], 0.0)
                        wbuf[b][r, pl.ds((4 + j) * 16, 16)] = jnp.maximum(
                            hi * svs[4 + j] + tvs[4 + j], 0.0)
                    return rr

                lax.fori_loop(0, KS, row_body, 0)
                pltpu.async_copy(wbuf[b], acc_shared.at[ibuf[b][il]],
                                 semw[b], add=True)

                @pl.when(c + 2 < NCS)
                def _next():
                    start_read(b, 1 - il, c + 2)

        return 0

    lax.fori_loop(0, (NCS + 3) // 4, chunk_body, 0)
    pltpu.make_async_copy(wbuf[0], acc_shared.at[ibuf[0][0]], semw[0]).wait()
    pltpu.make_async_copy(wbuf[1], acc_shared.at[ibuf[1][1]], semw[1]).wait()
    plsc.subcore_barrier()
    pltpu.sync_copy(acc_shared.at[pl.ds(sid * NPT, NPT)],
                    out_hbm.at[cid, pl.ds(sid * NPT, NPT)])

    @pl.when(sid == NS - 1)
    def _dump_tail():
        pltpu.sync_copy(acc_shared.at[pl.ds(NS * NPT, NTAIL)],
                        out_hbm.at[cid, pl.ds(NS * NPT, NTAIL)])


def _scatter_call(h3, eflat, sf, zeros_nd):
    mesh = plsc.VectorSubcoreMesh(core_axis_name="c", subcore_axis_name="s")
    f = functools.partial(
        pl.kernel,
        mesh=mesh,
        out_type=jax.ShapeDtypeStruct((NC, N, D), jnp.float32),
        scratch_types=[
            pltpu.VMEM((KS, DW), jnp.int32),
            pltpu.VMEM((KS, DW), jnp.int32),
            pltpu.VMEM((KS, D), jnp.float32),
            pltpu.VMEM((KS, D), jnp.float32),
            pltpu.VMEM((KS,), jnp.int32),
            pltpu.VMEM((KS,), jnp.int32),
            pltpu.VMEM((KS,), jnp.int32),
            pltpu.VMEM((KS,), jnp.int32),
            pltpu.VMEM((8, D), jnp.float32),
            pltpu.VMEM_SHARED((N, D), jnp.float32),
            pltpu.SemaphoreType.DMA,
            pltpu.SemaphoreType.DMA,
            pltpu.SemaphoreType.DMA,
            pltpu.SemaphoreType.DMA,
            pltpu.SemaphoreType.DMA,
            pltpu.SemaphoreType.DMA,
            pltpu.SemaphoreType.DMA,
            pltpu.SemaphoreType.DMA,
        ],
    )(_scatter_body)
    return f(h3, eflat, sf, zeros_nd)


def kernel(X, edge_index, W1, b1, g1, be1, W2, b2, g2, be2, W3, b3, g3, be3):
    eflat = edge_index.astype(jnp.int32).reshape(2 * E)

    y1, y2 = _y_call(X, W1, b1.reshape(1, D))
    h1, st1p = _gather_call(y1, y2, eflat)
    h2, st2 = _mlp1_call(h1, st1p, g1.reshape(1, D), be1.reshape(1, D),
                         W2, b2.reshape(1, D))
    h3, sf, _st3 = _mlp2_call(h2, st2, g2.reshape(1, D), be2.reshape(1, D),
                              W3, b3.reshape(1, D),
                              g3.reshape(1, D), be3.reshape(1, D))
    parts = _scatter_call(h3, eflat, sf, jnp.zeros((N, D), jnp.float32))
    return _add_call(parts)
